# Initial kernel scaffold; baseline (speedup 1.0000x reference)
import jax
import jax.numpy as jnp
from jax.experimental import pallas as pl


def kernel(x, edge_index, W1, att1, b1, W2, att2, b2):
    raise NotImplementedError("write your pallas kernel here")


# speedup vs baseline: 40.7316x; 40.7316x over previous
import functools

import jax
import jax.numpy as jnp
from jax import lax
from jax.experimental import pallas as pl
from jax.experimental.pallas import tpu as pltpu
from jax.experimental.pallas import tpu_sc as plsc

N = 10000
DIN = 128
H1 = 8
C1 = 16
D1 = 128
D2 = 64
E_RAW = 320000
E2 = E_RAW + N
CH = 128
NWORK = 32
K_CH = -(-E2 // (NWORK * CH))
TOT = NWORK * K_CH * CH
RPT = 8 * (-(-(N + 1) // (16 * 8)))
ROWS = RPT * 16
BN = 1000


def _k1_body(x_ref, w_ref, ai_ref, aj_ref, x1_out, ai_out, aj_out):
    x1 = jnp.dot(x_ref[...], w_ref[...], preferred_element_type=jnp.float32)
    x1_out[...] = x1
    ai_out[...] = jnp.dot(x1, ai_ref[...], preferred_element_type=jnp.float32)
    aj_out[...] = jnp.dot(x1, aj_ref[...], preferred_element_type=jnp.float32)


def _tc_stage1(x, W1, Ai, Aj):
    grid = (N // BN,)
    return pl.pallas_call(
        _k1_body,
        grid=grid,
        in_specs=[
            pl.BlockSpec((BN, DIN), lambda i: (i, 0)),
            pl.BlockSpec((DIN, D1), lambda i: (0, 0)),
            pl.BlockSpec((D1, 16), lambda i: (0, 0)),
            pl.BlockSpec((D1, 16), lambda i: (0, 0)),
        ],
        out_specs=[
            pl.BlockSpec((BN, D1), lambda i: (i, 0)),
            pl.BlockSpec((BN, 16), lambda i: (i, 0)),
            pl.BlockSpec((BN, 16), lambda i: (i, 0)),
        ],
        out_shape=[
            jax.ShapeDtypeStruct((N, D1), jnp.float32),
            jax.ShapeDtypeStruct((N, 16), jnp.float32),
            jax.ShapeDtypeStruct((N, 16), jnp.float32),
        ],
    )(x, W1, Ai, Aj)


def _k2_body(num_ref, den_ref, b1_ref, w2_ref, a2i_ref, a2j_ref, sel_ref,
             x2_out, ai2_out, aj2_out):
    num = num_ref[0] + num_ref[1]
    den = den_ref[0] + den_ref[1]
    denb = jnp.dot(den, sel_ref[...], preferred_element_type=jnp.float32)
    h = num / (denb + 1e-16) + b1_ref[...]
    h = jnp.where(h > 0, h, jnp.exp(jnp.minimum(h, 0.0)) - 1.0)
    x2 = jnp.dot(h, w2_ref[...], preferred_element_type=jnp.float32)
    x2_out[...] = x2
    ai2_out[...] = jnp.dot(x2, a2i_ref[...], preferred_element_type=jnp.float32)
    aj2_out[...] = jnp.dot(x2, a2j_ref[...], preferred_element_type=jnp.float32)


def _tc_stage2(num_parts, den_parts, b1, W2, A2i, A2j, SEL):
    grid = (N // BN,)
    return pl.pallas_call(
        _k2_body,
        grid=grid,
        in_specs=[
            pl.BlockSpec((2, BN, D1), lambda i: (0, i, 0)),
            pl.BlockSpec((2, BN, 16), lambda i: (0, i, 0)),
            pl.BlockSpec((1, D1), lambda i: (0, 0)),
            pl.BlockSpec((D1, D2), lambda i: (0, 0)),
            pl.BlockSpec((D2, 16), lambda i: (0, 0)),
            pl.BlockSpec((D2, 16), lambda i: (0, 0)),
            pl.BlockSpec((16, D1), lambda i: (0, 0)),
        ],
        out_specs=[
            pl.BlockSpec((BN, D2), lambda i: (i, 0)),
            pl.BlockSpec((BN, 16), lambda i: (i, 0)),
            pl.BlockSpec((BN, 16), lambda i: (i, 0)),
        ],
        out_shape=[
            jax.ShapeDtypeStruct((N, D2), jnp.float32),
            jax.ShapeDtypeStruct((N, 16), jnp.float32),
            jax.ShapeDtypeStruct((N, 16), jnp.float32),
        ],
    )(num_parts, den_parts, b1, W2, A2i, A2j, SEL)


def _k3_body(num_ref, den_ref, b2_ref, out_ref):
    num = num_ref[0] + num_ref[1]
    den = den_ref[0] + den_ref[1]
    out_ref[...] = num / (den[:, 0:1] + 1e-16) + b2_ref[...]


def _tc_stage3(num_parts, den_parts, b2):
    grid = (N // BN,)
    return pl.pallas_call(
        _k3_body,
        grid=grid,
        in_specs=[
            pl.BlockSpec((2, BN, D2), lambda i: (0, i, 0)),
            pl.BlockSpec((2, BN, 16), lambda i: (0, i, 0)),
            pl.BlockSpec((1, D2), lambda i: (0, 0)),
        ],
        out_specs=pl.BlockSpec((BN, D2), lambda i: (i, 0)),
        out_shape=jax.ShapeDtypeStruct((N, D2), jnp.float32),
    )(num_parts, den_parts, b2)


def _make_edge_kernel(D, heads_for_blocks):
    blocks = D // 16
    mesh = plsc.VectorSubcoreMesh(core_axis_name="c", subcore_axis_name="s")

    @functools.partial(
        pl.kernel,
        mesh=mesh,
        compiler_params=pltpu.CompilerParams(use_tc_tiling_on_sc=False),
        out_type=(
            jax.ShapeDtypeStruct((2, ROWS, D), jnp.float32),
            jax.ShapeDtypeStruct((2, ROWS, 16), jnp.float32),
        ),
        scratch_types=[
            pltpu.VMEM((CH,), jnp.int32),
            pltpu.VMEM((CH,), jnp.int32),
            pltpu.VMEM((CH,), jnp.int32),
            pltpu.VMEM((CH, 16), jnp.float32),
            pltpu.VMEM((CH, 16), jnp.float32),
            pltpu.VMEM((CH, D), jnp.float32),
            pltpu.VMEM((CH, 16), jnp.float32),
            pltpu.VMEM_SHARED((ROWS, D), jnp.float32),
            pltpu.VMEM_SHARED((ROWS, 16), jnp.float32),
            pltpu.SemaphoreType.DMA,
            pltpu.SemaphoreType.DMA,
            pltpu.SemaphoreType.DMA,
        ],
    )
    def edge_kernel(src_hbm, dsts_hbm, dstg_hbm, xtab, aitab, ajtab,
                    zd_hbm, z16_hbm, num_out, den_out,
                    src_v, dsts_v, dstg_v, ai_v, aj_v, x_v, w_v,
                    num_sh, den_sh, sem1, sem2, sem3):
        c = lax.axis_index("c")
        s = lax.axis_index("s")
        wid = c * 16 + s

        pltpu.sync_copy(zd_hbm, num_sh.at[pl.ds(s * RPT, RPT)])
        pltpu.sync_copy(z16_hbm, den_sh.at[pl.ds(s * RPT, RPT)])
        plsc.subcore_barrier()

        def chunk(k, carry):
            base = (wid * K_CH + k) * CH
            pltpu.sync_copy(src_hbm.at[pl.ds(base, CH)], src_v)
            pltpu.sync_copy(dsts_hbm.at[pl.ds(base, CH)], dsts_v)
            pltpu.sync_copy(dstg_hbm.at[pl.ds(base, CH)], dstg_v)
            cp1 = pltpu.async_copy(aitab.at[dstg_v], ai_v, sem1)
            cp2 = pltpu.async_copy(ajtab.at[src_v], aj_v, sem2)
            cp3 = pltpu.async_copy(xtab.at[src_v], x_v, sem3)
            cp1.wait()
            cp2.wait()
            cp3.wait()

            def edge(e, carry2):
                sv = ai_v[e, :] + aj_v[e, :]
                sv = jnp.where(sv > 0, sv, 0.2 * sv)
                wv = jnp.exp(sv)
                w_v[e, :] = wv
                for b in range(blocks):
                    ws = wv[heads_for_blocks[b]]
                    x_v[e, pl.ds(16 * b, 16)] = x_v[e, pl.ds(16 * b, 16)] * ws
                return carry2

            lax.fori_loop(0, CH, edge, 0)
            pltpu.sync_copy(x_v, num_sh.at[dsts_v], add=True)
            pltpu.sync_copy(w_v, den_sh.at[dsts_v], add=True)
            return carry

        lax.fori_loop(0, K_CH, chunk, 0)
        plsc.subcore_barrier()

        pltpu.sync_copy(num_sh.at[pl.ds(s * RPT, RPT)],
                        num_out.at[c, pl.ds(s * RPT, RPT)])
        pltpu.sync_copy(den_sh.at[pl.ds(s * RPT, RPT)],
                        den_out.at[c, pl.ds(s * RPT, RPT)])

    return edge_kernel


_make_edge_kernel = functools.lru_cache(maxsize=None)(_make_edge_kernel)


def kernel(x, edge_index, W1, att1, b1, W2, att2, b2):
    f32 = jnp.float32
    i32 = jnp.int32

    src, dst = edge_index[0], edge_index[1]
    keep = src != dst
    dst = jnp.where(keep, dst, N)
    loop = jnp.arange(N, dtype=i32)
    src_all = jnp.concatenate([src, loop])
    dst_all = jnp.concatenate([dst, loop])
    pad = TOT - E2
    src_all = jnp.concatenate([src_all, jnp.zeros((pad,), i32)])
    dst_scat = jnp.concatenate([dst_all, jnp.full((pad,), N, i32)])
    dst_gath = jnp.minimum(dst_scat, N - 1)

    atti = att1[0, :, :C1]
    attj = att1[0, :, C1:]
    eye = jnp.eye(H1, dtype=f32)
    Ai = (eye[:, None, :] * atti[:, :, None]).reshape(D1, H1)
    Aj = (eye[:, None, :] * attj[:, :, None]).reshape(D1, H1)
    Ai = jnp.pad(Ai, ((0, 0), (0, 16 - H1)))
    Aj = jnp.pad(Aj, ((0, 0), (0, 16 - H1)))
    A2i = jnp.broadcast_to(att2[0, 0, :D2][:, None], (D2, 16)).astype(f32)
    A2j = jnp.broadcast_to(att2[0, 0, D2:][:, None], (D2, 16)).astype(f32)
    SEL = jnp.kron(jnp.eye(H1, dtype=f32), jnp.ones((1, C1), f32))
    SEL = jnp.pad(SEL, ((0, 16 - H1), (0, 0)))

    z128 = jnp.zeros((RPT, D1), f32)
    z64 = jnp.zeros((RPT, D2), f32)
    z16 = jnp.zeros((RPT, 16), f32)

    x1, ai, aj = _tc_stage1(x, W1, Ai, Aj)
    num1, den1 = _make_edge_kernel(D1, tuple(range(H1)))(
        src_all, dst_scat, dst_gath, x1, ai, aj, z128, z16)
    x2, ai2, aj2 = _tc_stage2(num1, den1, b1.reshape(1, D1),
                              W2, A2i, A2j, SEL)
    num2, den2 = _make_edge_kernel(D2, (0, 0, 0, 0))(
        src_all, dst_scat, dst_gath, x2, ai2, aj2, z64, z16)
    return _tc_stage3(num2, den2, b2.reshape(1, D2))


# speedup vs baseline: 48.1417x; 1.1819x over previous
import functools

import jax
import jax.numpy as jnp
from jax import lax
from jax.experimental import pallas as pl
from jax.experimental.pallas import tpu as pltpu
from jax.experimental.pallas import tpu_sc as plsc

N = 10000
DIN = 128
H1 = 8
C1 = 16
D1 = 128
D2 = 64
E_RAW = 320000
E2 = E_RAW + N
CH = 96
NWORK = 32
K_CH = 2 * (-(-E2 // (NWORK * CH * 2)))
TOT = NWORK * K_CH * CH
RPT = 8 * (-(-(N + 1) // (16 * 8)))
ROWS = RPT * 16
BN = 1000


def _k1_body(x_ref, w_ref, ai_ref, aj_ref, x1_out, ai_out, aj_out):
    x1 = jnp.dot(x_ref[...], w_ref[...], preferred_element_type=jnp.float32)
    x1_out[...] = x1
    ai_out[...] = jnp.dot(x1, ai_ref[...], preferred_element_type=jnp.float32)
    aj_out[...] = jnp.dot(x1, aj_ref[...], preferred_element_type=jnp.float32)


def _tc_stage1(x, W1, Ai, Aj):
    grid = (N // BN,)
    return pl.pallas_call(
        _k1_body,
        grid=grid,
        in_specs=[
            pl.BlockSpec((BN, DIN), lambda i: (i, 0)),
            pl.BlockSpec((DIN, D1), lambda i: (0, 0)),
            pl.BlockSpec((D1, 16), lambda i: (0, 0)),
            pl.BlockSpec((D1, 16), lambda i: (0, 0)),
        ],
        out_specs=[
            pl.BlockSpec((BN, D1), lambda i: (i, 0)),
            pl.BlockSpec((BN, 16), lambda i: (i, 0)),
            pl.BlockSpec((BN, 16), lambda i: (i, 0)),
        ],
        out_shape=[
            jax.ShapeDtypeStruct((N, D1), jnp.float32),
            jax.ShapeDtypeStruct((N, 16), jnp.float32),
            jax.ShapeDtypeStruct((N, 16), jnp.float32),
        ],
    )(x, W1, Ai, Aj)


def _k2_body(num_ref, den_ref, b1_ref, w2_ref, a2i_ref, a2j_ref, sel_ref,
             x2_out, ai2_out, aj2_out):
    num = num_ref[0] + num_ref[1]
    den = den_ref[0] + den_ref[1]
    denb = jnp.dot(den, sel_ref[...], preferred_element_type=jnp.float32)
    h = num / (denb + 1e-16) + b1_ref[...]
    h = jnp.where(h > 0, h, jnp.exp(jnp.minimum(h, 0.0)) - 1.0)
    x2 = jnp.dot(h, w2_ref[...], preferred_element_type=jnp.float32)
    x2_out[...] = x2
    ai2_out[...] = jnp.dot(x2, a2i_ref[...], preferred_element_type=jnp.float32)
    aj2_out[...] = jnp.dot(x2, a2j_ref[...], preferred_element_type=jnp.float32)


def _tc_stage2(num_parts, den_parts, b1, W2, A2i, A2j, SEL):
    grid = (N // BN,)
    return pl.pallas_call(
        _k2_body,
        grid=grid,
        in_specs=[
            pl.BlockSpec((2, BN, D1), lambda i: (0, i, 0)),
            pl.BlockSpec((2, BN, 16), lambda i: (0, i, 0)),
            pl.BlockSpec((1, D1), lambda i: (0, 0)),
            pl.BlockSpec((D1, D2), lambda i: (0, 0)),
            pl.BlockSpec((D2, 16), lambda i: (0, 0)),
            pl.BlockSpec((D2, 16), lambda i: (0, 0)),
            pl.BlockSpec((16, D1), lambda i: (0, 0)),
        ],
        out_specs=[
            pl.BlockSpec((BN, D2), lambda i: (i, 0)),
            pl.BlockSpec((BN, 16), lambda i: (i, 0)),
            pl.BlockSpec((BN, 16), lambda i: (i, 0)),
        ],
        out_shape=[
            jax.ShapeDtypeStruct((N, D2), jnp.float32),
            jax.ShapeDtypeStruct((N, 16), jnp.float32),
            jax.ShapeDtypeStruct((N, 16), jnp.float32),
        ],
    )(num_parts, den_parts, b1, W2, A2i, A2j, SEL)


def _k3_body(num_ref, den_ref, b2_ref, out_ref):
    num = num_ref[0] + num_ref[1]
    den = den_ref[0] + den_ref[1]
    out_ref[...] = num / (den[:, 0:1] + 1e-16) + b2_ref[...]


def _tc_stage3(num_parts, den_parts, b2):
    grid = (N // BN,)
    return pl.pallas_call(
        _k3_body,
        grid=grid,
        in_specs=[
            pl.BlockSpec((2, BN, D2), lambda i: (0, i, 0)),
            pl.BlockSpec((2, BN, 16), lambda i: (0, i, 0)),
            pl.BlockSpec((1, D2), lambda i: (0, 0)),
        ],
        out_specs=pl.BlockSpec((BN, D2), lambda i: (i, 0)),
        out_shape=jax.ShapeDtypeStruct((N, D2), jnp.float32),
    )(num_parts, den_parts, b2)


def _make_edge_kernel(D, heads_for_blocks):
    blocks = D // 16
    mesh = plsc.VectorSubcoreMesh(core_axis_name="c", subcore_axis_name="s")

    @functools.partial(
        pl.kernel,
        mesh=mesh,
        compiler_params=pltpu.CompilerParams(use_tc_tiling_on_sc=False),
        out_type=(
            jax.ShapeDtypeStruct((2, ROWS, D), jnp.float32),
            jax.ShapeDtypeStruct((2, ROWS, 16), jnp.float32),
        ),
        scratch_types=(
            [pltpu.VMEM((CH,), jnp.int32)] * 6
            + [pltpu.VMEM((CH, 16), jnp.float32)] * 4
            + [pltpu.VMEM((CH, D), jnp.float32)] * 2
            + [pltpu.VMEM((CH, 16), jnp.float32)] * 2
            + [
                pltpu.VMEM_SHARED((ROWS, D), jnp.float32),
                pltpu.VMEM_SHARED((ROWS, 16), jnp.float32),
                pltpu.SemaphoreType.DMA,
                pltpu.SemaphoreType.DMA,
            ]
        ),
    )
    def edge_kernel(src_hbm, dsts_hbm, dstg_hbm, xtab, aitab, ajtab,
                    zd_hbm, z16_hbm, num_out, den_out,
                    src0, src1, dsts0, dsts1, dstg0, dstg1,
                    ai0, ai1, aj0, aj1, x0, x1, w0, w1,
                    num_sh, den_sh, sem0, sem1):
        c = lax.axis_index("c")
        s = lax.axis_index("s")
        wid = c * 16 + s
        bufs = (
            (src0, dsts0, dstg0, ai0, aj0, x0, w0, sem0),
            (src1, dsts1, dstg1, ai1, aj1, x1, w1, sem1),
        )

        pltpu.sync_copy(zd_hbm, num_sh.at[pl.ds(s * RPT, RPT)])
        pltpu.sync_copy(z16_hbm, den_sh.at[pl.ds(s * RPT, RPT)])
        plsc.subcore_barrier()

        def fire(k, buf):
            src_v, dsts_v, dstg_v, ai_v, aj_v, x_v, _, sem = buf
            base = (wid * K_CH + k) * CH
            pltpu.sync_copy(src_hbm.at[pl.ds(base, CH)], src_v)
            pltpu.sync_copy(dsts_hbm.at[pl.ds(base, CH)], dsts_v)
            pltpu.sync_copy(dstg_hbm.at[pl.ds(base, CH)], dstg_v)
            pltpu.async_copy(aitab.at[dstg_v], ai_v, sem)
            pltpu.async_copy(ajtab.at[src_v], aj_v, sem)
            pltpu.async_copy(xtab.at[src_v], x_v, sem)

        def consume(buf):
            src_v, dsts_v, dstg_v, ai_v, aj_v, x_v, w_v, sem = buf
            pltpu.make_async_copy(aitab.at[dstg_v], ai_v, sem).wait()
            pltpu.make_async_copy(ajtab.at[src_v], aj_v, sem).wait()
            pltpu.make_async_copy(xtab.at[src_v], x_v, sem).wait()

            def edge(e, carry2):
                sv = ai_v[e, :] + aj_v[e, :]
                sv = jnp.where(sv > 0, sv, 0.2 * sv)
                wv = jnp.exp(sv)
                w_v[e, :] = wv
                for b in range(blocks):
                    ws = wv[heads_for_blocks[b]]
                    x_v[e, pl.ds(16 * b, 16)] = x_v[e, pl.ds(16 * b, 16)] * ws
                return carry2

            lax.fori_loop(0, CH, edge, 0, unroll=4)
            pltpu.sync_copy(x_v, num_sh.at[dsts_v], add=True)
            pltpu.sync_copy(w_v, den_sh.at[dsts_v], add=True)

        fire(0, bufs[0])

        def pair(i, carry):
            k2 = i * 2
            for b in (0, 1):
                k = k2 + b

                @pl.when(k + 1 < K_CH)
                def _prefetch():
                    fire(k + 1, bufs[1 - b])

                consume(bufs[b])
            return carry

        lax.fori_loop(0, K_CH // 2, pair, 0)
        plsc.subcore_barrier()

        pltpu.sync_copy(num_sh.at[pl.ds(s * RPT, RPT)],
                        num_out.at[c, pl.ds(s * RPT, RPT)])
        pltpu.sync_copy(den_sh.at[pl.ds(s * RPT, RPT)],
                        den_out.at[c, pl.ds(s * RPT, RPT)])

    return edge_kernel


_make_edge_kernel = functools.lru_cache(maxsize=None)(_make_edge_kernel)


def kernel(x, edge_index, W1, att1, b1, W2, att2, b2):
    f32 = jnp.float32
    i32 = jnp.int32

    src, dst = edge_index[0], edge_index[1]
    keep = src != dst
    dst = jnp.where(keep, dst, N)
    loop = jnp.arange(N, dtype=i32)
    src_all = jnp.concatenate([src, loop])
    dst_all = jnp.concatenate([dst, loop])
    pad = TOT - E2
    src_all = jnp.concatenate([src_all, jnp.zeros((pad,), i32)])
    dst_scat = jnp.concatenate([dst_all, jnp.full((pad,), N, i32)])
    dst_gath = jnp.minimum(dst_scat, N - 1)

    atti = att1[0, :, :C1]
    attj = att1[0, :, C1:]
    eye = jnp.eye(H1, dtype=f32)
    Ai = (eye[:, None, :] * atti[:, :, None]).reshape(D1, H1)
    Aj = (eye[:, None, :] * attj[:, :, None]).reshape(D1, H1)
    Ai = jnp.pad(Ai, ((0, 0), (0, 16 - H1)))
    Aj = jnp.pad(Aj, ((0, 0), (0, 16 - H1)))
    A2i = jnp.broadcast_to(att2[0, 0, :D2][:, None], (D2, 16)).astype(f32)
    A2j = jnp.broadcast_to(att2[0, 0, D2:][:, None], (D2, 16)).astype(f32)
    SEL = jnp.kron(jnp.eye(H1, dtype=f32), jnp.ones((1, C1), f32))
    SEL = jnp.pad(SEL, ((0, 16 - H1), (0, 0)))

    z128 = jnp.zeros((RPT, D1), f32)
    z64 = jnp.zeros((RPT, D2), f32)
    z16 = jnp.zeros((RPT, 16), f32)

    x1, ai, aj = _tc_stage1(x, W1, Ai, Aj)
    num1, den1 = _make_edge_kernel(D1, tuple(range(H1)))(
        src_all, dst_scat, dst_gath, x1, ai, aj, z128, z16)
    x2, ai2, aj2 = _tc_stage2(num1, den1, b1.reshape(1, D1),
                              W2, A2i, A2j, SEL)
    num2, den2 = _make_edge_kernel(D2, (0, 0, 0, 0))(
        src_all, dst_scat, dst_gath, x2, ai2, aj2, z64, z16)
    return _tc_stage3(num2, den2, b2.reshape(1, D2))


# speedup vs baseline: 70.5640x; 1.4658x over previous
import functools

import jax
import jax.numpy as jnp
from jax import lax
from jax.experimental import pallas as pl
from jax.experimental.pallas import tpu as pltpu
from jax.experimental.pallas import tpu_sc as plsc

N = 10000
DIN = 128
H1 = 8
C1 = 16
D1 = 128
D2 = 64
E_RAW = 320000
E2 = E_RAW + N
CH = 96
NWORK = 32
K_CH = 2 * (-(-E2 // (NWORK * CH * 2)))
TOT = NWORK * K_CH * CH
RPT = 8 * (-(-(N + 1) // (16 * 8)))
ROWS = RPT * 16
BN = 1000


def _k1_body(x_ref, w_ref, ai_ref, aj_ref, x1_out, ai_out, aj_out):
    x1 = jnp.dot(x_ref[...], w_ref[...], preferred_element_type=jnp.float32)
    x1_out[...] = x1
    ai_out[...] = jnp.dot(x1, ai_ref[...], preferred_element_type=jnp.float32)
    aj_out[...] = jnp.dot(x1, aj_ref[...], preferred_element_type=jnp.float32)


def _tc_stage1(x, W1, Ai, Aj):
    grid = (N // BN,)
    return pl.pallas_call(
        _k1_body,
        grid=grid,
        in_specs=[
            pl.BlockSpec((BN, DIN), lambda i: (i, 0)),
            pl.BlockSpec((DIN, D1), lambda i: (0, 0)),
            pl.BlockSpec((D1, 16), lambda i: (0, 0)),
            pl.BlockSpec((D1, 16), lambda i: (0, 0)),
        ],
        out_specs=[
            pl.BlockSpec((BN, D1), lambda i: (i, 0)),
            pl.BlockSpec((BN, 16), lambda i: (i, 0)),
            pl.BlockSpec((BN, 16), lambda i: (i, 0)),
        ],
        out_shape=[
            jax.ShapeDtypeStruct((N, D1), jnp.float32),
            jax.ShapeDtypeStruct((N, 16), jnp.float32),
            jax.ShapeDtypeStruct((N, 16), jnp.float32),
        ],
    )(x, W1, Ai, Aj)


def _k2_body(num_ref, den_ref, b1_ref, w2_ref, a2i_ref, a2j_ref, sel_ref,
             x2_out, ai2_out, aj2_out):
    num = num_ref[0] + num_ref[1]
    den = den_ref[0] + den_ref[1]
    denb = jnp.dot(den, sel_ref[...], preferred_element_type=jnp.float32)
    h = num / (denb + 1e-16) + b1_ref[...]
    h = jnp.where(h > 0, h, jnp.exp(jnp.minimum(h, 0.0)) - 1.0)
    x2 = jnp.dot(h, w2_ref[...], preferred_element_type=jnp.float32)
    x2_out[...] = x2
    ai2_out[...] = jnp.dot(x2, a2i_ref[...], preferred_element_type=jnp.float32)
    aj2_out[...] = jnp.dot(x2, a2j_ref[...], preferred_element_type=jnp.float32)


def _tc_stage2(num_parts, den_parts, b1, W2, A2i, A2j, SEL):
    grid = (N // BN,)
    return pl.pallas_call(
        _k2_body,
        grid=grid,
        in_specs=[
            pl.BlockSpec((2, BN, D1), lambda i: (0, i, 0)),
            pl.BlockSpec((2, BN, 16), lambda i: (0, i, 0)),
            pl.BlockSpec((1, D1), lambda i: (0, 0)),
            pl.BlockSpec((D1, D2), lambda i: (0, 0)),
            pl.BlockSpec((D2, 16), lambda i: (0, 0)),
            pl.BlockSpec((D2, 16), lambda i: (0, 0)),
            pl.BlockSpec((16, D1), lambda i: (0, 0)),
        ],
        out_specs=[
            pl.BlockSpec((BN, D2), lambda i: (i, 0)),
            pl.BlockSpec((BN, 16), lambda i: (i, 0)),
            pl.BlockSpec((BN, 16), lambda i: (i, 0)),
        ],
        out_shape=[
            jax.ShapeDtypeStruct((N, D2), jnp.float32),
            jax.ShapeDtypeStruct((N, 16), jnp.float32),
            jax.ShapeDtypeStruct((N, 16), jnp.float32),
        ],
    )(num_parts, den_parts, b1, W2, A2i, A2j, SEL)


def _k3_body(num_ref, den_ref, b2_ref, out_ref):
    num = num_ref[0] + num_ref[1]
    den = den_ref[0] + den_ref[1]
    out_ref[...] = num / (den[:, 0:1] + 1e-16) + b2_ref[...]


def _tc_stage3(num_parts, den_parts, b2):
    grid = (N // BN,)
    return pl.pallas_call(
        _k3_body,
        grid=grid,
        in_specs=[
            pl.BlockSpec((2, BN, D2), lambda i: (0, i, 0)),
            pl.BlockSpec((2, BN, 16), lambda i: (0, i, 0)),
            pl.BlockSpec((1, D2), lambda i: (0, 0)),
        ],
        out_specs=pl.BlockSpec((BN, D2), lambda i: (i, 0)),
        out_shape=jax.ShapeDtypeStruct((N, D2), jnp.float32),
    )(num_parts, den_parts, b2)


def _make_edge_kernel(D, heads_for_blocks):
    blocks = D // 16
    mesh = plsc.VectorSubcoreMesh(core_axis_name="c", subcore_axis_name="s")

    @functools.partial(
        pl.kernel,
        mesh=mesh,
        compiler_params=pltpu.CompilerParams(use_tc_tiling_on_sc=False),
        out_type=(
            jax.ShapeDtypeStruct((2, ROWS, D), jnp.float32),
            jax.ShapeDtypeStruct((2, ROWS, 16), jnp.float32),
        ),
        scratch_types=(
            [pltpu.VMEM((CH,), jnp.int32)] * 6
            + [pltpu.VMEM((CH, 16), jnp.float32)] * 4
            + [pltpu.VMEM((CH, D), jnp.float32)] * 2
            + [pltpu.VMEM((CH, 16), jnp.float32)] * 2
            + [
                pltpu.VMEM_SHARED((ROWS, D), jnp.float32),
                pltpu.VMEM_SHARED((ROWS, 16), jnp.float32),
                pltpu.SemaphoreType.DMA,
                pltpu.SemaphoreType.DMA,
            ]
        ),
    )
    def edge_kernel(src_hbm, dsts_hbm, dstg_hbm, xtab, aitab, ajtab,
                    zd_hbm, z16_hbm, num_out, den_out,
                    src0, src1, dsts0, dsts1, dstg0, dstg1,
                    ai0, ai1, aj0, aj1, x0, x1, w0, w1,
                    num_sh, den_sh, sem0, sem1):
        c = lax.axis_index("c")
        s = lax.axis_index("s")
        wid = c * 16 + s
        bufs = (
            (src0, dsts0, dstg0, ai0, aj0, x0, w0, sem0),
            (src1, dsts1, dstg1, ai1, aj1, x1, w1, sem1),
        )

        pltpu.sync_copy(zd_hbm, num_sh.at[pl.ds(s * RPT, RPT)])
        pltpu.sync_copy(z16_hbm, den_sh.at[pl.ds(s * RPT, RPT)])
        plsc.subcore_barrier()

        def fire(k, buf):
            src_v, dsts_v, dstg_v, ai_v, aj_v, x_v, _, sem = buf
            base = (wid * K_CH + k) * CH
            pltpu.sync_copy(src_hbm.at[pl.ds(base, CH)], src_v)
            pltpu.sync_copy(dsts_hbm.at[pl.ds(base, CH)], dsts_v)
            pltpu.sync_copy(dstg_hbm.at[pl.ds(base, CH)], dstg_v)
            pltpu.async_copy(aitab.at[dstg_v], ai_v, sem)
            pltpu.async_copy(ajtab.at[src_v], aj_v, sem)
            pltpu.async_copy(xtab.at[src_v], x_v, sem)

        def consume(buf):
            src_v, dsts_v, dstg_v, ai_v, aj_v, x_v, w_v, sem = buf
            pltpu.make_async_copy(aitab.at[dstg_v], ai_v, sem).wait()
            pltpu.make_async_copy(ajtab.at[src_v], aj_v, sem).wait()
            pltpu.make_async_copy(xtab.at[src_v], x_v, sem).wait()

            @plsc.parallel_loop(0, CH, unroll=4)
            def _edge(e):
                sv = ai_v[e, :] + aj_v[e, :]
                sv = jnp.where(sv > 0, sv, 0.2 * sv)
                wv = jnp.exp(sv)
                w_v[e, :] = wv
                for b in range(blocks):
                    ws = wv[heads_for_blocks[b]]
                    x_v[e, pl.ds(16 * b, 16)] = x_v[e, pl.ds(16 * b, 16)] * ws
            pltpu.sync_copy(x_v, num_sh.at[dsts_v], add=True)
            pltpu.sync_copy(w_v, den_sh.at[dsts_v], add=True)

        fire(0, bufs[0])

        def pair(i, carry):
            k2 = i * 2
            for b in (0, 1):
                k = k2 + b

                @pl.when(k + 1 < K_CH)
                def _prefetch():
                    fire(k + 1, bufs[1 - b])

                consume(bufs[b])
            return carry

        lax.fori_loop(0, K_CH // 2, pair, 0)
        plsc.subcore_barrier()

        pltpu.sync_copy(num_sh.at[pl.ds(s * RPT, RPT)],
                        num_out.at[c, pl.ds(s * RPT, RPT)])
        pltpu.sync_copy(den_sh.at[pl.ds(s * RPT, RPT)],
                        den_out.at[c, pl.ds(s * RPT, RPT)])

    return edge_kernel


_make_edge_kernel = functools.lru_cache(maxsize=None)(_make_edge_kernel)


def kernel(x, edge_index, W1, att1, b1, W2, att2, b2):
    f32 = jnp.float32
    i32 = jnp.int32

    src, dst = edge_index[0], edge_index[1]
    keep = src != dst
    dst = jnp.where(keep, dst, N)
    loop = jnp.arange(N, dtype=i32)
    src_all = jnp.concatenate([src, loop])
    dst_all = jnp.concatenate([dst, loop])
    pad = TOT - E2
    src_all = jnp.concatenate([src_all, jnp.zeros((pad,), i32)])
    dst_scat = jnp.concatenate([dst_all, jnp.full((pad,), N, i32)])
    dst_gath = jnp.minimum(dst_scat, N - 1)

    atti = att1[0, :, :C1]
    attj = att1[0, :, C1:]
    eye = jnp.eye(H1, dtype=f32)
    Ai = (eye[:, None, :] * atti[:, :, None]).reshape(D1, H1)
    Aj = (eye[:, None, :] * attj[:, :, None]).reshape(D1, H1)
    Ai = jnp.pad(Ai, ((0, 0), (0, 16 - H1)))
    Aj = jnp.pad(Aj, ((0, 0), (0, 16 - H1)))
    A2i = jnp.broadcast_to(att2[0, 0, :D2][:, None], (D2, 16)).astype(f32)
    A2j = jnp.broadcast_to(att2[0, 0, D2:][:, None], (D2, 16)).astype(f32)
    SEL = jnp.kron(jnp.eye(H1, dtype=f32), jnp.ones((1, C1), f32))
    SEL = jnp.pad(SEL, ((0, 16 - H1), (0, 0)))

    z128 = jnp.zeros((RPT, D1), f32)
    z64 = jnp.zeros((RPT, D2), f32)
    z16 = jnp.zeros((RPT, 16), f32)

    x1, ai, aj = _tc_stage1(x, W1, Ai, Aj)
    num1, den1 = _make_edge_kernel(D1, tuple(range(H1)))(
        src_all, dst_scat, dst_gath, x1, ai, aj, z128, z16)
    x2, ai2, aj2 = _tc_stage2(num1, den1, b1.reshape(1, D1),
                              W2, A2i, A2j, SEL)
    num2, den2 = _make_edge_kernel(D2, (0, 0, 0, 0))(
        src_all, dst_scat, dst_gath, x2, ai2, aj2, z64, z16)
    return _tc_stage3(num2, den2, b2.reshape(1, D2))


# speedup vs baseline: 76.3268x; 1.0817x over previous
import functools

import jax
import jax.numpy as jnp
from jax import lax
from jax.experimental import pallas as pl
from jax.experimental.pallas import tpu as pltpu
from jax.experimental.pallas import tpu_sc as plsc

N = 10000
DIN = 128
H1 = 8
C1 = 16
D1 = 128
D2 = 64
E_RAW = 320000
E2 = E_RAW + N
CH = 96
NWORK = 32
K_CH = 2 * (-(-E2 // (NWORK * CH * 2)))
TOT = NWORK * K_CH * CH
RPT = 8 * (-(-(N + 1) // (16 * 8)))
ROWS = RPT * 16
BN = 1000


def _k1_body(x_ref, w_ref, ai_ref, aj_ref, x1_out, ai_out, aj_out):
    x1 = jnp.dot(x_ref[...], w_ref[...], preferred_element_type=jnp.float32)
    x1_out[...] = x1
    ai_out[...] = jnp.dot(x1, ai_ref[...], preferred_element_type=jnp.float32)
    aj_out[...] = jnp.dot(x1, aj_ref[...], preferred_element_type=jnp.float32)


def _tc_stage1(x, W1, Ai, Aj):
    grid = (N // BN,)
    return pl.pallas_call(
        _k1_body,
        grid=grid,
        in_specs=[
            pl.BlockSpec((BN, DIN), lambda i: (i, 0)),
            pl.BlockSpec((DIN, D1), lambda i: (0, 0)),
            pl.BlockSpec((D1, 16), lambda i: (0, 0)),
            pl.BlockSpec((D1, 16), lambda i: (0, 0)),
        ],
        out_specs=[
            pl.BlockSpec((BN, D1), lambda i: (i, 0)),
            pl.BlockSpec((BN, 16), lambda i: (i, 0)),
            pl.BlockSpec((BN, 16), lambda i: (i, 0)),
        ],
        out_shape=[
            jax.ShapeDtypeStruct((N, D1), jnp.float32),
            jax.ShapeDtypeStruct((N, 16), jnp.float32),
            jax.ShapeDtypeStruct((N, 16), jnp.float32),
        ],
    )(x, W1, Ai, Aj)


def _k2_body(num_ref, den_ref, b1_ref, w2_ref, a2i_ref, a2j_ref, sel_ref,
             x2_out, ai2_out, aj2_out):
    num = num_ref[0] + num_ref[1]
    den = den_ref[0] + den_ref[1]
    denb = jnp.dot(den, sel_ref[...], preferred_element_type=jnp.float32)
    h = num / (denb + 1e-16) + b1_ref[...]
    h = jnp.where(h > 0, h, jnp.exp(jnp.minimum(h, 0.0)) - 1.0)
    x2 = jnp.dot(h, w2_ref[...], preferred_element_type=jnp.float32)
    x2_out[...] = x2
    ai2_out[...] = jnp.dot(x2, a2i_ref[...], preferred_element_type=jnp.float32)
    aj2_out[...] = jnp.dot(x2, a2j_ref[...], preferred_element_type=jnp.float32)


def _tc_stage2(num_parts, den_parts, b1, W2, A2i, A2j, SEL):
    grid = (N // BN,)
    return pl.pallas_call(
        _k2_body,
        grid=grid,
        in_specs=[
            pl.BlockSpec((2, BN, D1), lambda i: (0, i, 0)),
            pl.BlockSpec((2, BN, 16), lambda i: (0, i, 0)),
            pl.BlockSpec((1, D1), lambda i: (0, 0)),
            pl.BlockSpec((D1, D2), lambda i: (0, 0)),
            pl.BlockSpec((D2, 16), lambda i: (0, 0)),
            pl.BlockSpec((D2, 16), lambda i: (0, 0)),
            pl.BlockSpec((16, D1), lambda i: (0, 0)),
        ],
        out_specs=[
            pl.BlockSpec((BN, D2), lambda i: (i, 0)),
            pl.BlockSpec((BN, 16), lambda i: (i, 0)),
            pl.BlockSpec((BN, 16), lambda i: (i, 0)),
        ],
        out_shape=[
            jax.ShapeDtypeStruct((N, D2), jnp.float32),
            jax.ShapeDtypeStruct((N, 16), jnp.float32),
            jax.ShapeDtypeStruct((N, 16), jnp.float32),
        ],
    )(num_parts, den_parts, b1, W2, A2i, A2j, SEL)


def _k3_body(num_ref, den_ref, b2_ref, out_ref):
    num = num_ref[0] + num_ref[1]
    den = den_ref[0] + den_ref[1]
    out_ref[...] = num / (den[:, 0:1] + 1e-16) + b2_ref[...]


def _tc_stage3(num_parts, den_parts, b2):
    grid = (N // BN,)
    return pl.pallas_call(
        _k3_body,
        grid=grid,
        in_specs=[
            pl.BlockSpec((2, BN, D2), lambda i: (0, i, 0)),
            pl.BlockSpec((2, BN, 16), lambda i: (0, i, 0)),
            pl.BlockSpec((1, D2), lambda i: (0, 0)),
        ],
        out_specs=pl.BlockSpec((BN, D2), lambda i: (i, 0)),
        out_shape=jax.ShapeDtypeStruct((N, D2), jnp.float32),
    )(num_parts, den_parts, b2)


def _make_edge_kernel(D, heads_for_blocks):
    blocks = D // 16
    mesh = plsc.VectorSubcoreMesh(core_axis_name="c", subcore_axis_name="s")

    @functools.partial(
        pl.kernel,
        mesh=mesh,
        compiler_params=pltpu.CompilerParams(use_tc_tiling_on_sc=False),
        out_type=(
            jax.ShapeDtypeStruct((2, ROWS, D), jnp.float32),
            jax.ShapeDtypeStruct((2, ROWS, 16), jnp.float32),
        ),
        scratch_types=(
            [pltpu.VMEM((CH,), jnp.int32)] * 6
            + [pltpu.VMEM((CH, 16), jnp.float32)] * 4
            + [pltpu.VMEM((CH, D), jnp.float32)] * 2
            + [pltpu.VMEM((CH, 16), jnp.float32)] * 2
            + [
                pltpu.VMEM_SHARED((ROWS, D), jnp.float32),
                pltpu.VMEM_SHARED((ROWS, 16), jnp.float32),
                pltpu.SemaphoreType.DMA,
                pltpu.SemaphoreType.DMA,
                pltpu.SemaphoreType.DMA,
                pltpu.SemaphoreType.DMA,
            ]
        ),
    )
    def edge_kernel(src_hbm, dsts_hbm, dstg_hbm, xtab, aitab, ajtab,
                    zd_hbm, z16_hbm, num_out, den_out,
                    src0, src1, dsts0, dsts1, dstg0, dstg1,
                    ai0, ai1, aj0, aj1, x0, x1, w0, w1,
                    num_sh, den_sh, semi0, semi1, semg0, semg1):
        c = lax.axis_index("c")
        s = lax.axis_index("s")
        wid = c * 16 + s
        bufs = (
            (src0, dsts0, dstg0, ai0, aj0, x0, w0, semi0, semg0),
            (src1, dsts1, dstg1, ai1, aj1, x1, w1, semi1, semg1),
        )

        pltpu.sync_copy(zd_hbm, num_sh.at[pl.ds(s * RPT, RPT)])
        pltpu.sync_copy(z16_hbm, den_sh.at[pl.ds(s * RPT, RPT)])
        plsc.subcore_barrier()

        def fire_idx(k, buf):
            src_v, dsts_v, dstg_v = buf[0], buf[1], buf[2]
            semi = buf[7]
            base = (wid * K_CH + k) * CH
            pltpu.async_copy(src_hbm.at[pl.ds(base, CH)], src_v, semi)
            pltpu.async_copy(dsts_hbm.at[pl.ds(base, CH)], dsts_v, semi)
            pltpu.async_copy(dstg_hbm.at[pl.ds(base, CH)], dstg_v, semi)

        def fire_gathers(k, buf):
            src_v, dsts_v, dstg_v, ai_v, aj_v, x_v, _, semi, semg = buf
            base = (wid * K_CH + k) * CH
            pltpu.make_async_copy(src_hbm.at[pl.ds(base, CH)], src_v, semi).wait()
            pltpu.make_async_copy(dsts_hbm.at[pl.ds(base, CH)], dsts_v, semi).wait()
            pltpu.make_async_copy(dstg_hbm.at[pl.ds(base, CH)], dstg_v, semi).wait()
            pltpu.async_copy(aitab.at[dstg_v], ai_v, semg)
            pltpu.async_copy(ajtab.at[src_v], aj_v, semg)
            pltpu.async_copy(xtab.at[src_v], x_v, semg)

        def consume(buf):
            src_v, dsts_v, dstg_v, ai_v, aj_v, x_v, w_v, semi, semg = buf
            pltpu.make_async_copy(aitab.at[dstg_v], ai_v, semg).wait()
            pltpu.make_async_copy(ajtab.at[src_v], aj_v, semg).wait()
            pltpu.make_async_copy(xtab.at[src_v], x_v, semg).wait()

            @plsc.parallel_loop(0, CH, unroll=8)
            def _edge(e):
                sv = ai_v[e, :] + aj_v[e, :]
                sv = jnp.where(sv > 0, sv, 0.2 * sv)
                wv = jnp.exp(sv)
                w_v[e, :] = wv
                for b in range(blocks):
                    ws = wv[heads_for_blocks[b]]
                    x_v[e, pl.ds(16 * b, 16)] = x_v[e, pl.ds(16 * b, 16)] * ws
            pltpu.sync_copy(x_v, num_sh.at[dsts_v], add=True)
            pltpu.sync_copy(w_v, den_sh.at[dsts_v], add=True)

        fire_idx(0, bufs[0])
        fire_gathers(0, bufs[0])

        def pair(i, carry):
            k2 = i * 2
            for b in (0, 1):
                k = k2 + b

                @pl.when(k + 1 < K_CH)
                def _prefetch():
                    fire_idx(k + 1, bufs[1 - b])
                    fire_gathers(k + 1, bufs[1 - b])

                consume(bufs[b])
            return carry

        lax.fori_loop(0, K_CH // 2, pair, 0)
        plsc.subcore_barrier()

        pltpu.sync_copy(num_sh.at[pl.ds(s * RPT, RPT)],
                        num_out.at[c, pl.ds(s * RPT, RPT)])
        pltpu.sync_copy(den_sh.at[pl.ds(s * RPT, RPT)],
                        den_out.at[c, pl.ds(s * RPT, RPT)])

    return edge_kernel


_make_edge_kernel = functools.lru_cache(maxsize=None)(_make_edge_kernel)


def kernel(x, edge_index, W1, att1, b1, W2, att2, b2):
    f32 = jnp.float32
    i32 = jnp.int32

    src, dst = edge_index[0], edge_index[1]
    keep = src != dst
    dst = jnp.where(keep, dst, N)
    loop = jnp.arange(N, dtype=i32)
    src_all = jnp.concatenate([src, loop])
    dst_all = jnp.concatenate([dst, loop])
    pad = TOT - E2
    src_all = jnp.concatenate([src_all, jnp.zeros((pad,), i32)])
    dst_scat = jnp.concatenate([dst_all, jnp.full((pad,), N, i32)])
    dst_gath = jnp.minimum(dst_scat, N - 1)

    atti = att1[0, :, :C1]
    attj = att1[0, :, C1:]
    eye = jnp.eye(H1, dtype=f32)
    Ai = (eye[:, None, :] * atti[:, :, None]).reshape(D1, H1)
    Aj = (eye[:, None, :] * attj[:, :, None]).reshape(D1, H1)
    Ai = jnp.pad(Ai, ((0, 0), (0, 16 - H1)))
    Aj = jnp.pad(Aj, ((0, 0), (0, 16 - H1)))
    A2i = jnp.broadcast_to(att2[0, 0, :D2][:, None], (D2, 16)).astype(f32)
    A2j = jnp.broadcast_to(att2[0, 0, D2:][:, None], (D2, 16)).astype(f32)
    SEL = jnp.kron(jnp.eye(H1, dtype=f32), jnp.ones((1, C1), f32))
    SEL = jnp.pad(SEL, ((0, 16 - H1), (0, 0)))

    z128 = jnp.zeros((RPT, D1), f32)
    z64 = jnp.zeros((RPT, D2), f32)
    z16 = jnp.zeros((RPT, 16), f32)

    x1, ai, aj = _tc_stage1(x, W1, Ai, Aj)
    num1, den1 = _make_edge_kernel(D1, tuple(range(H1)))(
        src_all, dst_scat, dst_gath, x1, ai, aj, z128, z16)
    x2, ai2, aj2 = _tc_stage2(num1, den1, b1.reshape(1, D1),
                              W2, A2i, A2j, SEL)
    num2, den2 = _make_edge_kernel(D2, (0, 0, 0, 0))(
        src_all, dst_scat, dst_gath, x2, ai2, aj2, z64, z16)
    return _tc_stage3(num2, den2, b2.reshape(1, D2))


# speedup vs baseline: 85.5873x; 1.1213x over previous
import functools

import jax
import jax.numpy as jnp
from jax import lax
from jax.experimental import pallas as pl
from jax.experimental.pallas import tpu as pltpu
from jax.experimental.pallas import tpu_sc as plsc

N = 10000
DIN = 128
H1 = 8
C1 = 16
D1 = 128
D2 = 64
E_RAW = 320000
E2 = E_RAW + N
CH = 96
NWORK = 32
K_CH = 2 * (-(-E2 // (NWORK * CH * 2)))
TOT = NWORK * K_CH * CH
RPT = 8 * (-(-(N + 1) // (16 * 8)))
ROWS = RPT * 16
BN = 1000


def _k1_body(x_ref, w_ref, ai_ref, aj_ref, x1_out, ai_out, aj_out):
    x1 = jnp.dot(x_ref[...], w_ref[...], preferred_element_type=jnp.float32)
    x1_out[...] = x1
    ai_out[...] = jnp.dot(x1, ai_ref[...], preferred_element_type=jnp.float32)
    aj_out[...] = jnp.dot(x1, aj_ref[...], preferred_element_type=jnp.float32)


def _tc_stage1(x, W1, Ai, Aj):
    grid = (N // BN,)
    return pl.pallas_call(
        _k1_body,
        grid=grid,
        in_specs=[
            pl.BlockSpec((BN, DIN), lambda i: (i, 0)),
            pl.BlockSpec((DIN, D1), lambda i: (0, 0)),
            pl.BlockSpec((D1, 16), lambda i: (0, 0)),
            pl.BlockSpec((D1, 16), lambda i: (0, 0)),
        ],
        out_specs=[
            pl.BlockSpec((BN, D1), lambda i: (i, 0)),
            pl.BlockSpec((BN, 16), lambda i: (i, 0)),
            pl.BlockSpec((BN, 16), lambda i: (i, 0)),
        ],
        out_shape=[
            jax.ShapeDtypeStruct((N, D1), jnp.float32),
            jax.ShapeDtypeStruct((N, 16), jnp.float32),
            jax.ShapeDtypeStruct((N, 16), jnp.float32),
        ],
    )(x, W1, Ai, Aj)


def _k2_body(num_ref, den_ref, b1_ref, w2_ref, a2i_ref, a2j_ref, sel_ref,
             x2_out, ai2_out, aj2_out):
    num = num_ref[0] + num_ref[1]
    den = den_ref[0] + den_ref[1]
    denb = jnp.dot(den, sel_ref[...], preferred_element_type=jnp.float32)
    h = num / (denb + 1e-16) + b1_ref[...]
    h = jnp.where(h > 0, h, jnp.exp(jnp.minimum(h, 0.0)) - 1.0)
    x2 = jnp.dot(h, w2_ref[...], preferred_element_type=jnp.float32)
    x2_out[...] = x2
    ai2_out[...] = jnp.dot(x2, a2i_ref[...], preferred_element_type=jnp.float32)
    aj2_out[...] = jnp.dot(x2, a2j_ref[...], preferred_element_type=jnp.float32)


def _tc_stage2(num_parts, den_parts, b1, W2, A2i, A2j, SEL):
    grid = (N // BN,)
    return pl.pallas_call(
        _k2_body,
        grid=grid,
        in_specs=[
            pl.BlockSpec((2, BN, D1), lambda i: (0, i, 0)),
            pl.BlockSpec((2, BN, 16), lambda i: (0, i, 0)),
            pl.BlockSpec((1, D1), lambda i: (0, 0)),
            pl.BlockSpec((D1, D2), lambda i: (0, 0)),
            pl.BlockSpec((D2, 16), lambda i: (0, 0)),
            pl.BlockSpec((D2, 16), lambda i: (0, 0)),
            pl.BlockSpec((16, D1), lambda i: (0, 0)),
        ],
        out_specs=[
            pl.BlockSpec((BN, D2), lambda i: (i, 0)),
            pl.BlockSpec((BN, 16), lambda i: (i, 0)),
            pl.BlockSpec((BN, 16), lambda i: (i, 0)),
        ],
        out_shape=[
            jax.ShapeDtypeStruct((N, D2), jnp.float32),
            jax.ShapeDtypeStruct((N, 16), jnp.float32),
            jax.ShapeDtypeStruct((N, 16), jnp.float32),
        ],
    )(num_parts, den_parts, b1, W2, A2i, A2j, SEL)


def _k3_body(num_ref, den_ref, b2_ref, out_ref):
    num = num_ref[0] + num_ref[1]
    den = den_ref[0] + den_ref[1]
    out_ref[...] = num / (den[:, 0:1] + 1e-16) + b2_ref[...]


def _tc_stage3(num_parts, den_parts, b2):
    grid = (N // BN,)
    return pl.pallas_call(
        _k3_body,
        grid=grid,
        in_specs=[
            pl.BlockSpec((2, BN, D2), lambda i: (0, i, 0)),
            pl.BlockSpec((2, BN, 16), lambda i: (0, i, 0)),
            pl.BlockSpec((1, D2), lambda i: (0, 0)),
        ],
        out_specs=pl.BlockSpec((BN, D2), lambda i: (i, 0)),
        out_shape=jax.ShapeDtypeStruct((N, D2), jnp.float32),
    )(num_parts, den_parts, b2)


def _make_edge_kernel(D, heads_for_blocks):
    blocks = D // 16
    mesh = plsc.VectorSubcoreMesh(core_axis_name="c", subcore_axis_name="s")

    @functools.partial(
        pl.kernel,
        mesh=mesh,
        compiler_params=pltpu.CompilerParams(use_tc_tiling_on_sc=False),
        out_type=(
            jax.ShapeDtypeStruct((2, ROWS, D), jnp.float32),
            jax.ShapeDtypeStruct((2, ROWS, 16), jnp.float32),
        ),
        scratch_types=(
            [pltpu.VMEM((CH,), jnp.int32)] * 6
            + [pltpu.VMEM((CH, 16), jnp.float32)] * 4
            + [pltpu.VMEM((CH, D), jnp.float32)] * 2
            + [pltpu.VMEM((CH, 16), jnp.float32)] * 2
            + [
                pltpu.VMEM_SHARED((ROWS, D), jnp.float32),
                pltpu.VMEM_SHARED((ROWS, 16), jnp.float32),
                pltpu.SemaphoreType.DMA,
                pltpu.SemaphoreType.DMA,
                pltpu.SemaphoreType.DMA,
                pltpu.SemaphoreType.DMA,
            ]
        ),
    )
    def edge_kernel(src_hbm, dsts_hbm, dstg_hbm, xtab, aitab, ajtab,
                    zd_hbm, z16_hbm, num_out, den_out,
                    src0, src1, dsts0, dsts1, dstg0, dstg1,
                    ai0, ai1, aj0, aj1, x0, x1, w0, w1,
                    num_sh, den_sh, semi0, semi1, semg0, semg1):
        c = lax.axis_index("c")
        s = lax.axis_index("s")
        wid = c * 16 + s
        bufs = (
            (src0, dsts0, dstg0, ai0, aj0, x0, w0, semi0, semg0),
            (src1, dsts1, dstg1, ai1, aj1, x1, w1, semi1, semg1),
        )

        pltpu.sync_copy(zd_hbm, num_sh.at[pl.ds(s * RPT, RPT)])
        pltpu.sync_copy(z16_hbm, den_sh.at[pl.ds(s * RPT, RPT)])
        plsc.subcore_barrier()

        def fire_idx(k, buf):
            src_v, dsts_v, dstg_v = buf[0], buf[1], buf[2]
            semi = buf[7]
            base = (wid * K_CH + k) * CH
            pltpu.async_copy(src_hbm.at[pl.ds(base, CH)], src_v, semi)
            pltpu.async_copy(dsts_hbm.at[pl.ds(base, CH)], dsts_v, semi)
            pltpu.async_copy(dstg_hbm.at[pl.ds(base, CH)], dstg_v, semi)

        def fire_gathers(k, buf):
            src_v, dsts_v, dstg_v, ai_v, aj_v, x_v, _, semi, semg = buf
            base = (wid * K_CH + k) * CH
            pltpu.make_async_copy(src_hbm.at[pl.ds(base, CH)], src_v, semi).wait()
            pltpu.make_async_copy(dsts_hbm.at[pl.ds(base, CH)], dsts_v, semi).wait()
            pltpu.make_async_copy(dstg_hbm.at[pl.ds(base, CH)], dstg_v, semi).wait()
            pltpu.async_copy(aitab.at[dstg_v], ai_v, semg)
            pltpu.async_copy(ajtab.at[src_v], aj_v, semg)
            pltpu.async_copy(xtab.at[src_v], x_v, semg)

        def consume(buf):
            src_v, dsts_v, dstg_v, ai_v, aj_v, x_v, w_v, semi, semg = buf
            pltpu.make_async_copy(aitab.at[dstg_v], ai_v, semg).wait()
            pltpu.make_async_copy(ajtab.at[src_v], aj_v, semg).wait()
            pltpu.make_async_copy(xtab.at[src_v], x_v, semg).wait()

            @plsc.parallel_loop(0, CH, unroll=16)
            def _edge(e):
                sv = ai_v[e, :] + aj_v[e, :]
                sv = jnp.where(sv > 0, sv, 0.2 * sv)
                wv = jnp.exp(sv)
                w_v[e, :] = wv
                for b in range(blocks):
                    ws = wv[heads_for_blocks[b]]
                    x_v[e, pl.ds(16 * b, 16)] = x_v[e, pl.ds(16 * b, 16)] * ws
            pltpu.sync_copy(x_v, num_sh.at[dsts_v], add=True)
            pltpu.sync_copy(w_v, den_sh.at[dsts_v], add=True)

        fire_idx(0, bufs[0])
        fire_gathers(0, bufs[0])

        def pair(i, carry):
            k2 = i * 2
            for b in (0, 1):
                k = k2 + b

                @pl.when(k + 1 < K_CH)
                def _prefetch():
                    fire_idx(k + 1, bufs[1 - b])
                    fire_gathers(k + 1, bufs[1 - b])

                consume(bufs[b])
            return carry

        lax.fori_loop(0, K_CH // 2, pair, 0)
        plsc.subcore_barrier()

        pltpu.sync_copy(num_sh.at[pl.ds(s * RPT, RPT)],
                        num_out.at[c, pl.ds(s * RPT, RPT)])
        pltpu.sync_copy(den_sh.at[pl.ds(s * RPT, RPT)],
                        den_out.at[c, pl.ds(s * RPT, RPT)])

    return edge_kernel


_make_edge_kernel = functools.lru_cache(maxsize=None)(_make_edge_kernel)


def kernel(x, edge_index, W1, att1, b1, W2, att2, b2):
    f32 = jnp.float32
    i32 = jnp.int32

    src, dst = edge_index[0], edge_index[1]
    keep = src != dst
    dst = jnp.where(keep, dst, N)
    loop = jnp.arange(N, dtype=i32)
    src_all = jnp.concatenate([src, loop])
    dst_all = jnp.concatenate([dst, loop])
    pad = TOT - E2
    src_all = jnp.concatenate([src_all, jnp.zeros((pad,), i32)])
    dst_scat = jnp.concatenate([dst_all, jnp.full((pad,), N, i32)])
    dst_gath = jnp.minimum(dst_scat, N - 1)

    atti = att1[0, :, :C1]
    attj = att1[0, :, C1:]
    eye = jnp.eye(H1, dtype=f32)
    Ai = (eye[:, None, :] * atti[:, :, None]).reshape(D1, H1)
    Aj = (eye[:, None, :] * attj[:, :, None]).reshape(D1, H1)
    Ai = jnp.pad(Ai, ((0, 0), (0, 16 - H1)))
    Aj = jnp.pad(Aj, ((0, 0), (0, 16 - H1)))
    A2i = jnp.broadcast_to(att2[0, 0, :D2][:, None], (D2, 16)).astype(f32)
    A2j = jnp.broadcast_to(att2[0, 0, D2:][:, None], (D2, 16)).astype(f32)
    SEL = jnp.kron(jnp.eye(H1, dtype=f32), jnp.ones((1, C1), f32))
    SEL = jnp.pad(SEL, ((0, 16 - H1), (0, 0)))

    z128 = jnp.zeros((RPT, D1), f32)
    z64 = jnp.zeros((RPT, D2), f32)
    z16 = jnp.zeros((RPT, 16), f32)

    x1, ai, aj = _tc_stage1(x, W1, Ai, Aj)
    num1, den1 = _make_edge_kernel(D1, tuple(range(H1)))(
        src_all, dst_scat, dst_gath, x1, ai, aj, z128, z16)
    x2, ai2, aj2 = _tc_stage2(num1, den1, b1.reshape(1, D1),
                              W2, A2i, A2j, SEL)
    num2, den2 = _make_edge_kernel(D2, (0, 0, 0, 0))(
        src_all, dst_scat, dst_gath, x2, ai2, aj2, z64, z16)
    return _tc_stage3(num2, den2, b2.reshape(1, D2))


# speedup vs baseline: 85.7739x; 1.0022x over previous
import functools

import jax
import jax.numpy as jnp
from jax import lax
from jax.experimental import pallas as pl
from jax.experimental.pallas import tpu as pltpu
from jax.experimental.pallas import tpu_sc as plsc

N = 10000
DIN = 128
H1 = 8
C1 = 16
D1 = 128
D2 = 64
E_RAW = 320000
E2 = E_RAW + N
CH = 96
NWORK = 32
K_CH = 2 * (-(-E2 // (NWORK * CH * 2)))
TOT = NWORK * K_CH * CH
RPT = 8 * (-(-(N + 1) // (16 * 8)))
ROWS = RPT * 16
BN = 1000


def _k1_body(x_ref, w_ref, ai_ref, aj_ref, x1_out, ai_out, aj_out):
    x1 = jnp.dot(x_ref[...], w_ref[...], preferred_element_type=jnp.float32)
    x1_out[...] = x1
    ai_out[...] = jnp.dot(x1, ai_ref[...], preferred_element_type=jnp.float32)
    aj_out[...] = jnp.dot(x1, aj_ref[...], preferred_element_type=jnp.float32)


def _tc_stage1(x, W1, Ai, Aj):
    grid = (N // BN,)
    return pl.pallas_call(
        _k1_body,
        grid=grid,
        in_specs=[
            pl.BlockSpec((BN, DIN), lambda i: (i, 0)),
            pl.BlockSpec((DIN, D1), lambda i: (0, 0)),
            pl.BlockSpec((D1, 16), lambda i: (0, 0)),
            pl.BlockSpec((D1, 16), lambda i: (0, 0)),
        ],
        out_specs=[
            pl.BlockSpec((BN, D1), lambda i: (i, 0)),
            pl.BlockSpec((BN, 16), lambda i: (i, 0)),
            pl.BlockSpec((BN, 16), lambda i: (i, 0)),
        ],
        out_shape=[
            jax.ShapeDtypeStruct((N, D1), jnp.float32),
            jax.ShapeDtypeStruct((N, 16), jnp.float32),
            jax.ShapeDtypeStruct((N, 16), jnp.float32),
        ],
    )(x, W1, Ai, Aj)


def _k2_body(num_ref, den_ref, b1_ref, w2_ref, a2i_ref, a2j_ref, sel_ref,
             x2_out, ai2_out, aj2_out):
    num = num_ref[0] + num_ref[1]
    den = den_ref[0] + den_ref[1]
    denb = jnp.dot(den, sel_ref[...], preferred_element_type=jnp.float32)
    h = num / (denb + 1e-16) + b1_ref[...]
    h = jnp.where(h > 0, h, jnp.exp(jnp.minimum(h, 0.0)) - 1.0)
    x2 = jnp.dot(h, w2_ref[...], preferred_element_type=jnp.float32)
    x2_out[...] = x2
    ai2_out[...] = jnp.dot(x2, a2i_ref[...], preferred_element_type=jnp.float32)
    aj2_out[...] = jnp.dot(x2, a2j_ref[...], preferred_element_type=jnp.float32)


def _tc_stage2(num_parts, den_parts, b1, W2, A2i, A2j, SEL):
    grid = (N // BN,)
    return pl.pallas_call(
        _k2_body,
        grid=grid,
        in_specs=[
            pl.BlockSpec((2, BN, D1), lambda i: (0, i, 0)),
            pl.BlockSpec((2, BN, 16), lambda i: (0, i, 0)),
            pl.BlockSpec((1, D1), lambda i: (0, 0)),
            pl.BlockSpec((D1, D2), lambda i: (0, 0)),
            pl.BlockSpec((D2, 16), lambda i: (0, 0)),
            pl.BlockSpec((D2, 16), lambda i: (0, 0)),
            pl.BlockSpec((16, D1), lambda i: (0, 0)),
        ],
        out_specs=[
            pl.BlockSpec((BN, D2), lambda i: (i, 0)),
            pl.BlockSpec((BN, 16), lambda i: (i, 0)),
            pl.BlockSpec((BN, 16), lambda i: (i, 0)),
        ],
        out_shape=[
            jax.ShapeDtypeStruct((N, D2), jnp.float32),
            jax.ShapeDtypeStruct((N, 16), jnp.float32),
            jax.ShapeDtypeStruct((N, 16), jnp.float32),
        ],
    )(num_parts, den_parts, b1, W2, A2i, A2j, SEL)


def _k3_body(num_ref, den_ref, b2_ref, out_ref):
    num = num_ref[0] + num_ref[1]
    den = den_ref[0] + den_ref[1]
    out_ref[...] = num / (den[:, 0:1] + 1e-16) + b2_ref[...]


def _tc_stage3(num_parts, den_parts, b2):
    grid = (N // BN,)
    return pl.pallas_call(
        _k3_body,
        grid=grid,
        in_specs=[
            pl.BlockSpec((2, BN, D2), lambda i: (0, i, 0)),
            pl.BlockSpec((2, BN, 16), lambda i: (0, i, 0)),
            pl.BlockSpec((1, D2), lambda i: (0, 0)),
        ],
        out_specs=pl.BlockSpec((BN, D2), lambda i: (i, 0)),
        out_shape=jax.ShapeDtypeStruct((N, D2), jnp.float32),
    )(num_parts, den_parts, b2)


def _make_edge_kernel(D, heads_for_blocks):
    blocks = D // 16
    mesh = plsc.VectorSubcoreMesh(core_axis_name="c", subcore_axis_name="s")

    @functools.partial(
        pl.kernel,
        mesh=mesh,
        compiler_params=pltpu.CompilerParams(use_tc_tiling_on_sc=False),
        out_type=(
            jax.ShapeDtypeStruct((2, ROWS, D), jnp.float32),
            jax.ShapeDtypeStruct((2, ROWS, 16), jnp.float32),
        ),
        scratch_types=(
            [pltpu.VMEM((CH,), jnp.int32)] * 6
            + [pltpu.VMEM((CH, 16), jnp.float32)] * 4
            + [pltpu.VMEM((CH, D), jnp.float32)] * 2
            + [pltpu.VMEM((CH, 16), jnp.float32)] * 2
            + [
                pltpu.VMEM_SHARED((ROWS, D), jnp.float32),
                pltpu.VMEM_SHARED((ROWS, 16), jnp.float32),
                pltpu.SemaphoreType.DMA,
                pltpu.SemaphoreType.DMA,
                pltpu.SemaphoreType.DMA,
                pltpu.SemaphoreType.DMA,
            ]
        ),
    )
    def edge_kernel(src_hbm, dsts_hbm, dstg_hbm, xtab, aitab, ajtab,
                    zd_hbm, z16_hbm, num_out, den_out,
                    src0, src1, dsts0, dsts1, dstg0, dstg1,
                    ai0, ai1, aj0, aj1, x0, x1, w0, w1,
                    num_sh, den_sh, semi0, semi1, semg0, semg1):
        c = lax.axis_index("c")
        s = lax.axis_index("s")
        wid = c * 16 + s
        bufs = (
            (src0, dsts0, dstg0, ai0, aj0, x0, w0, semi0, semg0),
            (src1, dsts1, dstg1, ai1, aj1, x1, w1, semi1, semg1),
        )

        pltpu.sync_copy(zd_hbm, num_sh.at[pl.ds(s * RPT, RPT)])
        pltpu.sync_copy(z16_hbm, den_sh.at[pl.ds(s * RPT, RPT)])
        plsc.subcore_barrier()

        def fire_idx(k, buf):
            src_v, dsts_v, dstg_v = buf[0], buf[1], buf[2]
            semi = buf[7]
            base = (wid * K_CH + k) * CH
            pltpu.async_copy(src_hbm.at[pl.ds(base, CH)], src_v, semi)
            pltpu.async_copy(dsts_hbm.at[pl.ds(base, CH)], dsts_v, semi)
            pltpu.async_copy(dstg_hbm.at[pl.ds(base, CH)], dstg_v, semi)

        def fire_gathers(k, buf):
            src_v, dsts_v, dstg_v, ai_v, aj_v, x_v, _, semi, semg = buf
            base = (wid * K_CH + k) * CH
            pltpu.make_async_copy(src_hbm.at[pl.ds(base, CH)], src_v, semi).wait()
            pltpu.make_async_copy(dsts_hbm.at[pl.ds(base, CH)], dsts_v, semi).wait()
            pltpu.make_async_copy(dstg_hbm.at[pl.ds(base, CH)], dstg_v, semi).wait()
            pltpu.async_copy(aitab.at[dstg_v], ai_v, semg)
            pltpu.async_copy(ajtab.at[src_v], aj_v, semg)
            pltpu.async_copy(xtab.at[src_v], x_v, semg)

        def consume(buf):
            src_v, dsts_v, dstg_v, ai_v, aj_v, x_v, w_v, semi, semg = buf
            pltpu.make_async_copy(aitab.at[dstg_v], ai_v, semg).wait()
            pltpu.make_async_copy(ajtab.at[src_v], aj_v, semg).wait()
            pltpu.make_async_copy(xtab.at[src_v], x_v, semg).wait()

            @plsc.parallel_loop(0, CH, unroll=32)
            def _edge(e):
                sv = ai_v[e, :] + aj_v[e, :]
                sv = jnp.where(sv > 0, sv, 0.2 * sv)
                wv = jnp.exp(sv)
                w_v[e, :] = wv
                for b in range(blocks):
                    ws = wv[heads_for_blocks[b]]
                    x_v[e, pl.ds(16 * b, 16)] = x_v[e, pl.ds(16 * b, 16)] * ws
            pltpu.sync_copy(x_v, num_sh.at[dsts_v], add=True)
            pltpu.sync_copy(w_v, den_sh.at[dsts_v], add=True)

        fire_idx(0, bufs[0])
        fire_gathers(0, bufs[0])

        def pair(i, carry):
            k2 = i * 2
            for b in (0, 1):
                k = k2 + b

                @pl.when(k + 1 < K_CH)
                def _prefetch():
                    fire_idx(k + 1, bufs[1 - b])
                    fire_gathers(k + 1, bufs[1 - b])

                consume(bufs[b])
            return carry

        lax.fori_loop(0, K_CH // 2, pair, 0)
        plsc.subcore_barrier()

        pltpu.sync_copy(num_sh.at[pl.ds(s * RPT, RPT)],
                        num_out.at[c, pl.ds(s * RPT, RPT)])
        pltpu.sync_copy(den_sh.at[pl.ds(s * RPT, RPT)],
                        den_out.at[c, pl.ds(s * RPT, RPT)])

    return edge_kernel


_make_edge_kernel = functools.lru_cache(maxsize=None)(_make_edge_kernel)


def kernel(x, edge_index, W1, att1, b1, W2, att2, b2):
    f32 = jnp.float32
    i32 = jnp.int32

    src, dst = edge_index[0], edge_index[1]
    keep = src != dst
    dst = jnp.where(keep, dst, N)
    loop = jnp.arange(N, dtype=i32)
    src_all = jnp.concatenate([src, loop])
    dst_all = jnp.concatenate([dst, loop])
    pad = TOT - E2
    src_all = jnp.concatenate([src_all, jnp.zeros((pad,), i32)])
    dst_scat = jnp.concatenate([dst_all, jnp.full((pad,), N, i32)])
    dst_gath = jnp.minimum(dst_scat, N - 1)

    atti = att1[0, :, :C1]
    attj = att1[0, :, C1:]
    eye = jnp.eye(H1, dtype=f32)
    Ai = (eye[:, None, :] * atti[:, :, None]).reshape(D1, H1)
    Aj = (eye[:, None, :] * attj[:, :, None]).reshape(D1, H1)
    Ai = jnp.pad(Ai, ((0, 0), (0, 16 - H1)))
    Aj = jnp.pad(Aj, ((0, 0), (0, 16 - H1)))
    A2i = jnp.broadcast_to(att2[0, 0, :D2][:, None], (D2, 16)).astype(f32)
    A2j = jnp.broadcast_to(att2[0, 0, D2:][:, None], (D2, 16)).astype(f32)
    SEL = jnp.kron(jnp.eye(H1, dtype=f32), jnp.ones((1, C1), f32))
    SEL = jnp.pad(SEL, ((0, 16 - H1), (0, 0)))

    z128 = jnp.zeros((RPT, D1), f32)
    z64 = jnp.zeros((RPT, D2), f32)
    z16 = jnp.zeros((RPT, 16), f32)

    x1, ai, aj = _tc_stage1(x, W1, Ai, Aj)
    num1, den1 = _make_edge_kernel(D1, tuple(range(H1)))(
        src_all, dst_scat, dst_gath, x1, ai, aj, z128, z16)
    x2, ai2, aj2 = _tc_stage2(num1, den1, b1.reshape(1, D1),
                              W2, A2i, A2j, SEL)
    num2, den2 = _make_edge_kernel(D2, (0, 0, 0, 0))(
        src_all, dst_scat, dst_gath, x2, ai2, aj2, z64, z16)
    return _tc_stage3(num2, den2, b2.reshape(1, D2))
